# Initial kernel scaffold; baseline (speedup 1.0000x reference)
import jax
import jax.numpy as jnp
from jax.experimental import pallas as pl


def kernel(x, edge_index, batch, lin_W0, lin_b0, agg_W0, agg_b0, lin_W1, lin_b1, agg_W1, agg_b1, mp_W1, mp_b1, mp_W2, mp_b2):
    raise NotImplementedError("write your pallas kernel here")


# speedup vs baseline: 4.1266x; 4.1266x over previous
import jax
import jax.numpy as jnp
from jax import lax
from jax.experimental import pallas as pl
from jax.experimental.pallas import tpu as pltpu
from jax.experimental.pallas import tpu_sc as plsc

N_NODES = 10000
N_EDGES = 320000
D = 128
DH = D // 2

NC = 2
NS = 16

NP = 10240
NODE_ROWS_PER_TILE = NP // NS

EP = 327680
IDX_ROWS = EP // 128
ROWS_PER_TILE = IDX_ROWS // NS
CHUNK_ROWS = 4
N_CHUNKS = ROWS_PER_TILE // CHUNK_ROWS


def _make_agg_kernel(with_count):
  mesh = plsc.VectorSubcoreMesh(core_axis_name="c", subcore_axis_name="s",
                                num_cores=NC, num_subcores=NS)
  out_type = [jax.ShapeDtypeStruct((NC, NP, DH), jnp.float32)]
  scratch = [
      pltpu.VMEM((CHUNK_ROWS, 128), jnp.int32),
      pltpu.VMEM((CHUNK_ROWS, 128), jnp.int32),
      pltpu.VMEM((CHUNK_ROWS * 128, DH), jnp.float32),
      pltpu.VMEM_SHARED((NP, DH), jnp.float32),
      pltpu.SemaphoreType.DMA,
  ]
  if with_count:
    out_type.append(jax.ShapeDtypeStruct((NC, NP, 16), jnp.float32))
    scratch += [
        pltpu.VMEM((128, 16), jnp.float32),
        pltpu.VMEM_SHARED((NP, 16), jnp.float32),
    ]

  def body(x_hbm, src_hbm, dst_hbm, zrow_hbm, zcnt_hbm, ones_hbm,
           sum_out, cnt_out, src_v, dst_v, rows_v, acc_sh, sem,
           ones_v, cnt_sh):
    c = lax.axis_index("c")
    s = lax.axis_index("s")
    nb = s * NODE_ROWS_PER_TILE
    pltpu.sync_copy(zrow_hbm, acc_sh.at[pl.ds(nb, NODE_ROWS_PER_TILE)])
    if with_count:
      pltpu.sync_copy(zcnt_hbm, cnt_sh.at[pl.ds(nb, NODE_ROWS_PER_TILE)])
      pltpu.sync_copy(ones_hbm, ones_v)
    plsc.subcore_barrier()

    base = s * ROWS_PER_TILE

    @pl.loop(0, N_CHUNKS)
    def _chunk(k):
      r0 = base + k * CHUNK_ROWS
      pltpu.sync_copy(src_hbm.at[pl.ds(r0, CHUNK_ROWS)], src_v)
      pltpu.sync_copy(dst_hbm.at[pl.ds(r0, CHUNK_ROWS)], dst_v)
      descs = []
      for j in range(CHUNK_ROWS):
        descs.append(pltpu.async_copy(
            x_hbm.at[c].at[src_v.at[j]],
            rows_v.at[pl.ds(j * 128, 128)], sem))
      for d in descs:
        d.wait()
      for j in range(CHUNK_ROWS):
        pltpu.sync_copy(rows_v.at[pl.ds(j * 128, 128)],
                        acc_sh.at[dst_v.at[j]], add=True)
      if with_count:
        count_here = jnp.logical_or(
            jnp.logical_and(c == 0, k < N_CHUNKS // 2),
            jnp.logical_and(c != 0, k >= N_CHUNKS // 2))

        @pl.when(count_here)
        def _():
          for j in range(CHUNK_ROWS):
            pltpu.sync_copy(ones_v, cnt_sh.at[dst_v.at[j]], add=True)

    plsc.subcore_barrier()
    pltpu.sync_copy(acc_sh.at[pl.ds(nb, NODE_ROWS_PER_TILE)],
                    sum_out.at[c, pl.ds(nb, NODE_ROWS_PER_TILE)])
    if with_count:
      pltpu.sync_copy(cnt_sh.at[pl.ds(nb, NODE_ROWS_PER_TILE)],
                      cnt_out.at[c, pl.ds(nb, NODE_ROWS_PER_TILE)])

  if with_count:
    def body_wc(x_hbm, src_hbm, dst_hbm, zrow_hbm, zcnt_hbm, ones_hbm,
                sum_out, cnt_out, src_v, dst_v, rows_v, acc_sh, sem,
                ones_v, cnt_sh):
      body(x_hbm, src_hbm, dst_hbm, zrow_hbm, zcnt_hbm, ones_hbm,
           sum_out, cnt_out, src_v, dst_v, rows_v, acc_sh, sem,
           ones_v, cnt_sh)
    return pl.kernel(body_wc, out_type=tuple(out_type), mesh=mesh,
                     scratch_types=tuple(scratch),
                     compiler_params=pltpu.CompilerParams(
                         use_tc_tiling_on_sc=False))
  else:
    def body_nc(x_hbm, src_hbm, dst_hbm, zrow_hbm,
                sum_out, src_v, dst_v, rows_v, acc_sh, sem):
      body(x_hbm, src_hbm, dst_hbm, zrow_hbm, None, None,
           sum_out, None, src_v, dst_v, rows_v, acc_sh, sem, None, None)
    return pl.kernel(body_nc, out_type=out_type[0], mesh=mesh,
                     scratch_types=tuple(scratch),
                     compiler_params=pltpu.CompilerParams(
                         use_tc_tiling_on_sc=False))


_agg_cache = {}


def _agg(with_count):
  if with_count not in _agg_cache:
    _agg_cache[with_count] = _make_agg_kernel(with_count)
  return _agg_cache[with_count]


BLK = 2048


def _layer_common(p_ref, cnt_ref, x_ref, aW_ref, ab_ref, lW_ref, lb_ref):
  summed = jnp.concatenate([p_ref[0], p_ref[1]], axis=-1)
  cnt = cnt_ref[0][:, :1] + cnt_ref[1][:, :1]
  agg = summed / jnp.maximum(cnt, 1.0)
  x = jnp.concatenate([x_ref[0], x_ref[1]], axis=-1)
  t = jnp.maximum(
      jnp.dot(agg, aW_ref[...], preferred_element_type=jnp.float32)
      + ab_ref[...], 0.0)
  t = t + jnp.dot(x, lW_ref[...],
                  preferred_element_type=jnp.float32) + lb_ref[...]
  nrm = jnp.sqrt(jnp.sum(t * t, axis=-1, keepdims=True))
  t = t / jnp.maximum(nrm, 1e-12)
  return jnp.maximum(t, 0.0)


def _tc_layer1_body(p_ref, cnt_ref, x_ref, aW_ref, ab_ref, lW_ref, lb_ref,
                    o_ref):
  t = _layer_common(p_ref, cnt_ref, x_ref, aW_ref, ab_ref, lW_ref, lb_ref)
  o_ref[0] = t[:, :DH]
  o_ref[1] = t[:, DH:]


def _tc_layer2_body(p_ref, cnt_ref, x_ref, aW_ref, ab_ref, lW_ref, lb_ref,
                    m1W_ref, m1b_ref, m2W_ref, m2b_ref, o_ref):
  h = _layer_common(p_ref, cnt_ref, x_ref, aW_ref, ab_ref, lW_ref, lb_ref)
  h = jnp.dot(h, m1W_ref[...], preferred_element_type=jnp.float32) \
      + m1b_ref[...]
  h = jnp.dot(h, m2W_ref[...], preferred_element_type=jnp.float32) \
      + m2b_ref[...]
  m = jnp.max(h, axis=-1, keepdims=True)
  e = jnp.exp(h - m)
  o_ref[...] = (h - m) - jnp.log(jnp.sum(e, axis=-1, keepdims=True))


def _full_spec(shape):
  return pl.BlockSpec(shape, lambda i: tuple(0 for _ in shape))


_COMMON_SPECS = [
    pl.BlockSpec((NC, BLK, DH), lambda i: (0, i, 0)),
    pl.BlockSpec((NC, BLK, 16), lambda i: (0, i, 0)),
    pl.BlockSpec((NC, BLK, DH), lambda i: (0, i, 0)),
    _full_spec((D, D)), _full_spec((1, D)),
    _full_spec((D, D)), _full_spec((1, D)),
]

_tc_layer1 = pl.pallas_call(
    _tc_layer1_body,
    grid=(NP // BLK,),
    in_specs=_COMMON_SPECS,
    out_specs=pl.BlockSpec((NC, BLK, DH), lambda i: (0, i, 0)),
    out_shape=jax.ShapeDtypeStruct((NC, NP, DH), jnp.float32),
)

_tc_layer2 = pl.pallas_call(
    _tc_layer2_body,
    grid=(NP // BLK,),
    in_specs=_COMMON_SPECS + [
        _full_spec((D, D)), _full_spec((1, D)),
        _full_spec((D, 64)), _full_spec((1, 64)),
    ],
    out_specs=pl.BlockSpec((BLK, 64), lambda i: (i, 0)),
    out_shape=jax.ShapeDtypeStruct((NP, 64), jnp.float32),
)


def kernel(x, edge_index, batch, lin_W0, lin_b0, agg_W0, agg_b0,
           lin_W1, lin_b1, agg_W1, agg_b1, mp_W1, mp_b1, mp_W2, mp_b2):
  src = edge_index[0]
  dst = edge_index[1]

  xp = jnp.concatenate(
      [x, jnp.zeros((NP - N_NODES, D), jnp.float32)], axis=0)
  xs = jnp.moveaxis(xp.reshape(NP, NC, DH), 1, 0)
  pad = EP - N_EDGES
  srcR = jnp.concatenate([src, jnp.zeros((pad,), jnp.int32)]).reshape(
      IDX_ROWS, 128)
  dstR = jnp.concatenate(
      [dst, jnp.full((pad,), NP - 1, jnp.int32)]).reshape(IDX_ROWS, 128)
  ones = jnp.ones((128, 16), jnp.float32)
  zrow = jnp.zeros((NODE_ROWS_PER_TILE, DH), jnp.float32)
  zcnt = jnp.zeros((NODE_ROWS_PER_TILE, 16), jnp.float32)

  sum0, cnt = _agg(True)(xs, srcR, dstR, zrow, zcnt, ones)
  h1 = _tc_layer1(sum0, cnt, xs, agg_W0, agg_b0.reshape(1, D),
                  lin_W0, lin_b0.reshape(1, D))
  sum1 = _agg(False)(h1, srcR, dstR, zrow)
  out = _tc_layer2(sum1, cnt, h1, agg_W1, agg_b1.reshape(1, D),
                   lin_W1, lin_b1.reshape(1, D),
                   mp_W1, mp_b1.reshape(1, D),
                   mp_W2, mp_b2.reshape(1, 64))
  return out[:N_NODES]


# speedup vs baseline: 4.2528x; 1.0306x over previous
import jax
import jax.numpy as jnp
from jax import lax
from jax.experimental import pallas as pl
from jax.experimental.pallas import tpu as pltpu
from jax.experimental.pallas import tpu_sc as plsc

N_NODES = 10000
N_EDGES = 320000
D = 128
DH = D // 2

NC = 2
NS = 16

NP = 10240
NODE_ROWS_PER_TILE = NP // NS

EP = 327680
IDX_ROWS = EP // 128
ROWS_PER_TILE = IDX_ROWS // NS
CHUNK_ROWS = 4
N_CHUNKS = ROWS_PER_TILE // CHUNK_ROWS


def _make_agg_kernel():
  mesh = plsc.VectorSubcoreMesh(core_axis_name="c", subcore_axis_name="s",
                                num_cores=NC, num_subcores=NS)
  scratch = [
      pltpu.VMEM((2 * CHUNK_ROWS, 128), jnp.int32),
      pltpu.VMEM((2 * CHUNK_ROWS, 128), jnp.int32),
      pltpu.VMEM((2 * CHUNK_ROWS * 128, DH), jnp.float32),
      pltpu.VMEM_SHARED((NP, DH), jnp.float32),
      pltpu.SemaphoreType.DMA,
      pltpu.SemaphoreType.DMA,
  ]

  def body(x_hbm, src_hbm, dst_hbm, zrow_hbm,
           sum_out, src_v, dst_v, rows_v, acc_sh, sem_g, sem_s):
    c = lax.axis_index("c")
    s = lax.axis_index("s")
    nb = s * NODE_ROWS_PER_TILE
    pltpu.sync_copy(zrow_hbm, acc_sh.at[pl.ds(nb, NODE_ROWS_PER_TILE)])
    plsc.subcore_barrier()

    base = s * ROWS_PER_TILE

    def _row_buf(b, j):
      return rows_v.at[pl.ds((b * CHUNK_ROWS + j) * 128, 128)]

    def _idx_load(k, b):
      r0 = base + k * CHUNK_ROWS
      pltpu.sync_copy(src_hbm.at[pl.ds(r0, CHUNK_ROWS)],
                      src_v.at[pl.ds(b * CHUNK_ROWS, CHUNK_ROWS)])
      pltpu.sync_copy(dst_hbm.at[pl.ds(r0, CHUNK_ROWS)],
                      dst_v.at[pl.ds(b * CHUNK_ROWS, CHUNK_ROWS)])

    def _gather(k, b, issue):
      for j in range(CHUNK_ROWS):
        src = x_hbm.at[c].at[src_v.at[b * CHUNK_ROWS + j]]
        if issue:
          pltpu.async_copy(src, _row_buf(b, j), sem_g)
        else:
          pltpu.make_async_copy(src, _row_buf(b, j), sem_g).wait()

    def _scatter(k, b, issue):
      for j in range(CHUNK_ROWS):
        idx = dst_v.at[b * CHUNK_ROWS + j]
        if issue:
          pltpu.async_copy(_row_buf(b, j), acc_sh.at[idx], sem_s, add=True)
        else:
          pltpu.make_async_copy(_row_buf(b, j), acc_sh.at[idx], sem_s).wait()

    _idx_load(0, 0)
    _gather(0, 0, True)

    @pl.loop(0, N_CHUNKS, step=2)
    def _pipe(g):
      _idx_load(g + 1, 1)
      _gather(g, 0, False)
      _scatter(g, 0, True)

      @pl.when(g > 0)
      def _():
        _scatter(g - 1, 1, False)

      _gather(g + 1, 1, True)
      _gather(g + 1, 1, False)
      _scatter(g + 1, 1, True)
      _scatter(g, 0, False)

      @pl.when(g + 2 < N_CHUNKS)
      def _():
        _idx_load(g + 2, 0)
        _gather(g + 2, 0, True)

    _scatter(N_CHUNKS - 1, 1, False)
    plsc.subcore_barrier()
    pltpu.sync_copy(acc_sh.at[pl.ds(nb, NODE_ROWS_PER_TILE)],
                    sum_out.at[c, pl.ds(nb, NODE_ROWS_PER_TILE)])

  return pl.kernel(body, out_type=jax.ShapeDtypeStruct((NC, NP, DH),
                                                       jnp.float32),
                   mesh=mesh, scratch_types=tuple(scratch),
                   compiler_params=pltpu.CompilerParams(
                       use_tc_tiling_on_sc=False))


CNT_ROWS_PER_TILE = IDX_ROWS // (NC * NS)


def _make_count_kernel():
  mesh = plsc.VectorSubcoreMesh(core_axis_name="c", subcore_axis_name="s",
                                num_cores=NC, num_subcores=NS)
  scratch = [
      pltpu.VMEM((CNT_ROWS_PER_TILE, 128), jnp.int32),
      pltpu.VMEM((128, 16), jnp.float32),
      pltpu.VMEM_SHARED((NP, 16), jnp.float32),
      pltpu.SemaphoreType.DMA,
  ]

  def body(dst_hbm, zcnt_hbm, ones_hbm, cnt_out, dst_v, ones_v, cnt_sh,
           sem):
    c = lax.axis_index("c")
    s = lax.axis_index("s")
    nb = s * NODE_ROWS_PER_TILE
    pltpu.sync_copy(zcnt_hbm, cnt_sh.at[pl.ds(nb, NODE_ROWS_PER_TILE)])
    pltpu.sync_copy(ones_hbm, ones_v)
    plsc.subcore_barrier()

    base = (c * NS + s) * CNT_ROWS_PER_TILE
    pltpu.sync_copy(dst_hbm.at[pl.ds(base, CNT_ROWS_PER_TILE)], dst_v)

    @pl.loop(0, CNT_ROWS_PER_TILE, step=2)
    def _row(r):
      pltpu.async_copy(ones_v, cnt_sh.at[dst_v.at[r]], sem, add=True)
      pltpu.async_copy(ones_v, cnt_sh.at[dst_v.at[r + 1]], sem, add=True)
      pltpu.make_async_copy(ones_v, cnt_sh.at[dst_v.at[r]], sem).wait()
      pltpu.make_async_copy(ones_v, cnt_sh.at[dst_v.at[r + 1]], sem).wait()

    plsc.subcore_barrier()
    pltpu.sync_copy(cnt_sh.at[pl.ds(nb, NODE_ROWS_PER_TILE)],
                    cnt_out.at[c, pl.ds(nb, NODE_ROWS_PER_TILE)])

  return pl.kernel(body, out_type=jax.ShapeDtypeStruct((NC, NP, 16),
                                                       jnp.float32),
                   mesh=mesh, scratch_types=tuple(scratch),
                   compiler_params=pltpu.CompilerParams(
                       use_tc_tiling_on_sc=False))


_sc_cache = {}


def _agg():
  if 'agg' not in _sc_cache:
    _sc_cache['agg'] = _make_agg_kernel()
  return _sc_cache['agg']


def _count():
  if 'cnt' not in _sc_cache:
    _sc_cache['cnt'] = _make_count_kernel()
  return _sc_cache['cnt']


BLK = 2048


def _layer_common(p_ref, cnt_ref, x_ref, aW_ref, ab_ref, lW_ref, lb_ref):
  summed = jnp.concatenate([p_ref[0], p_ref[1]], axis=-1)
  cnt = cnt_ref[0][:, :1] + cnt_ref[1][:, :1]
  agg = summed / jnp.maximum(cnt, 1.0)
  x = jnp.concatenate([x_ref[0], x_ref[1]], axis=-1)
  t = jnp.maximum(
      jnp.dot(agg, aW_ref[...], preferred_element_type=jnp.float32)
      + ab_ref[...], 0.0)
  t = t + jnp.dot(x, lW_ref[...],
                  preferred_element_type=jnp.float32) + lb_ref[...]
  nrm = jnp.sqrt(jnp.sum(t * t, axis=-1, keepdims=True))
  t = t / jnp.maximum(nrm, 1e-12)
  return jnp.maximum(t, 0.0)


def _tc_layer_body(p_ref, cnt_ref, x_ref, aW_ref, ab_ref, lW_ref, lb_ref,
                   o_ref):
  t = _layer_common(p_ref, cnt_ref, x_ref, aW_ref, ab_ref, lW_ref, lb_ref)
  o_ref[0] = t[:, :DH]
  o_ref[1] = t[:, DH:]


def _tc_mlp_body(x_ref, m1W_ref, m1b_ref, m2W_ref, m2b_ref, o_ref):
  h = jnp.concatenate([x_ref[0], x_ref[1]], axis=-1)
  h = jnp.dot(h, m1W_ref[...], preferred_element_type=jnp.float32) \
      + m1b_ref[...]
  h = jnp.dot(h, m2W_ref[...], preferred_element_type=jnp.float32) \
      + m2b_ref[...]
  m = jnp.max(h, axis=-1, keepdims=True)
  e = jnp.exp(h - m)
  o_ref[...] = (h - m) - jnp.log(jnp.sum(e, axis=-1, keepdims=True))


def _full_spec(shape):
  return pl.BlockSpec(shape, lambda i: tuple(0 for _ in shape))


_tc_layer = pl.pallas_call(
    _tc_layer_body,
    grid=(NP // BLK,),
    in_specs=[
        pl.BlockSpec((NC, BLK, DH), lambda i: (0, i, 0)),
        pl.BlockSpec((NC, BLK, 16), lambda i: (0, i, 0)),
        pl.BlockSpec((NC, BLK, DH), lambda i: (0, i, 0)),
        _full_spec((D, D)), _full_spec((1, D)),
        _full_spec((D, D)), _full_spec((1, D)),
    ],
    out_specs=pl.BlockSpec((NC, BLK, DH), lambda i: (0, i, 0)),
    out_shape=jax.ShapeDtypeStruct((NC, NP, DH), jnp.float32),
)

_tc_mlp = pl.pallas_call(
    _tc_mlp_body,
    grid=(NP // BLK,),
    in_specs=[
        pl.BlockSpec((NC, BLK, DH), lambda i: (0, i, 0)),
        _full_spec((D, D)), _full_spec((1, D)),
        _full_spec((D, 64)), _full_spec((1, 64)),
    ],
    out_specs=pl.BlockSpec((BLK, 64), lambda i: (i, 0)),
    out_shape=jax.ShapeDtypeStruct((NP, 64), jnp.float32),
)


def kernel(x, edge_index, batch, lin_W0, lin_b0, agg_W0, agg_b0,
           lin_W1, lin_b1, agg_W1, agg_b1, mp_W1, mp_b1, mp_W2, mp_b2):
  src = edge_index[0]
  dst = edge_index[1]

  xp = jnp.concatenate(
      [x, jnp.zeros((NP - N_NODES, D), jnp.float32)], axis=0)
  xs = jnp.moveaxis(xp.reshape(NP, NC, DH), 1, 0)
  pad = EP - N_EDGES
  srcR = jnp.concatenate([src, jnp.zeros((pad,), jnp.int32)]).reshape(
      IDX_ROWS, 128)
  dstR = jnp.concatenate(
      [dst, jnp.full((pad,), NP - 1, jnp.int32)]).reshape(IDX_ROWS, 128)
  zrow = jnp.zeros((NODE_ROWS_PER_TILE, DH), jnp.float32)
  zcnt = jnp.zeros((NODE_ROWS_PER_TILE, 16), jnp.float32)
  ones = jnp.ones((128, 16), jnp.float32)

  cnt = _count()(dstR, zcnt, ones)

  aggW = jnp.stack([agg_W0, agg_W1])
  aggB = jnp.stack([agg_b0.reshape(1, D), agg_b1.reshape(1, D)])
  linW = jnp.stack([lin_W0, lin_W1])
  linB = jnp.stack([lin_b0.reshape(1, D), lin_b1.reshape(1, D)])

  def layer(i, h):
    sums = _agg()(h, srcR, dstR, zrow)
    return _tc_layer(sums, cnt, h, aggW[i], aggB[i], linW[i], linB[i])

  h = lax.fori_loop(0, 2, layer, xs)

  out = _tc_mlp(h, mp_W1, mp_b1.reshape(1, D), mp_W2, mp_b2.reshape(1, 64))
  return out[:N_NODES]


# speedup vs baseline: 4.2729x; 1.0047x over previous
import jax
import jax.numpy as jnp
from jax import lax
from jax.experimental import pallas as pl
from jax.experimental.pallas import tpu as pltpu
from jax.experimental.pallas import tpu_sc as plsc

N_NODES = 10000
N_EDGES = 320000
D = 128
DH = D // 2

NC = 2
NS = 16

NP = 10240
NODE_ROWS_PER_TILE = NP // NS

EP = 327680
IDX_ROWS = EP // 128
ROWS_PER_TILE = IDX_ROWS // NS
CHUNK_ROWS = 5
N_CHUNKS = ROWS_PER_TILE // CHUNK_ROWS


def _make_agg_kernel():
  mesh = plsc.VectorSubcoreMesh(core_axis_name="c", subcore_axis_name="s",
                                num_cores=NC, num_subcores=NS)
  scratch = [
      pltpu.VMEM((2 * CHUNK_ROWS, 128), jnp.int32),
      pltpu.VMEM((2 * CHUNK_ROWS, 128), jnp.int32),
      pltpu.VMEM((2 * CHUNK_ROWS * 128, DH), jnp.float32),
      pltpu.VMEM_SHARED((NP, DH), jnp.float32),
      pltpu.SemaphoreType.DMA,
      pltpu.SemaphoreType.DMA,
  ]

  def body(x_hbm, src_hbm, dst_hbm, zrow_hbm,
           sum_out, src_v, dst_v, rows_v, acc_sh, sem_g, sem_s):
    c = lax.axis_index("c")
    s = lax.axis_index("s")
    nb = s * NODE_ROWS_PER_TILE
    pltpu.sync_copy(zrow_hbm, acc_sh.at[pl.ds(nb, NODE_ROWS_PER_TILE)])
    plsc.subcore_barrier()

    base = s * ROWS_PER_TILE

    def _row_buf(b, j):
      return rows_v.at[pl.ds((b * CHUNK_ROWS + j) * 128, 128)]

    def _idx_load(k, b):
      r0 = base + k * CHUNK_ROWS
      pltpu.sync_copy(src_hbm.at[pl.ds(r0, CHUNK_ROWS)],
                      src_v.at[pl.ds(b * CHUNK_ROWS, CHUNK_ROWS)])
      pltpu.sync_copy(dst_hbm.at[pl.ds(r0, CHUNK_ROWS)],
                      dst_v.at[pl.ds(b * CHUNK_ROWS, CHUNK_ROWS)])

    def _gather(k, b, issue):
      for j in range(CHUNK_ROWS):
        src = x_hbm.at[c].at[src_v.at[b * CHUNK_ROWS + j]]
        if issue:
          pltpu.async_copy(src, _row_buf(b, j), sem_g)
        else:
          pltpu.make_async_copy(src, _row_buf(b, j), sem_g).wait()

    def _scatter(k, b, issue):
      for j in range(CHUNK_ROWS):
        idx = dst_v.at[b * CHUNK_ROWS + j]
        if issue:
          pltpu.async_copy(_row_buf(b, j), acc_sh.at[idx], sem_s, add=True)
        else:
          pltpu.make_async_copy(_row_buf(b, j), acc_sh.at[idx], sem_s).wait()

    _idx_load(0, 0)
    _gather(0, 0, True)

    @pl.loop(0, N_CHUNKS, step=2)
    def _pipe(g):
      _idx_load(g + 1, 1)
      _gather(g, 0, False)
      _scatter(g, 0, True)

      @pl.when(g > 0)
      def _():
        _scatter(g - 1, 1, False)

      _gather(g + 1, 1, True)
      _gather(g + 1, 1, False)
      _scatter(g + 1, 1, True)
      _scatter(g, 0, False)

      @pl.when(g + 2 < N_CHUNKS)
      def _():
        _idx_load(g + 2, 0)
        _gather(g + 2, 0, True)

    _scatter(N_CHUNKS - 1, 1, False)
    plsc.subcore_barrier()
    pltpu.sync_copy(acc_sh.at[pl.ds(nb, NODE_ROWS_PER_TILE)],
                    sum_out.at[c, pl.ds(nb, NODE_ROWS_PER_TILE)])

  return pl.kernel(body, out_type=jax.ShapeDtypeStruct((NC, NP, DH),
                                                       jnp.float32),
                   mesh=mesh, scratch_types=tuple(scratch),
                   compiler_params=pltpu.CompilerParams(
                       use_tc_tiling_on_sc=False))


CNT_ROWS_PER_TILE = IDX_ROWS // (NC * NS)


def _make_count_kernel():
  mesh = plsc.VectorSubcoreMesh(core_axis_name="c", subcore_axis_name="s",
                                num_cores=NC, num_subcores=NS)
  scratch = [
      pltpu.VMEM((CNT_ROWS_PER_TILE, 128), jnp.int32),
      pltpu.VMEM((128, 16), jnp.float32),
      pltpu.VMEM_SHARED((NP, 16), jnp.float32),
      pltpu.SemaphoreType.DMA,
  ]

  def body(dst_hbm, zcnt_hbm, ones_hbm, cnt_out, dst_v, ones_v, cnt_sh,
           sem):
    c = lax.axis_index("c")
    s = lax.axis_index("s")
    nb = s * NODE_ROWS_PER_TILE
    pltpu.sync_copy(zcnt_hbm, cnt_sh.at[pl.ds(nb, NODE_ROWS_PER_TILE)])
    pltpu.sync_copy(ones_hbm, ones_v)
    plsc.subcore_barrier()

    base = (c * NS + s) * CNT_ROWS_PER_TILE
    pltpu.sync_copy(dst_hbm.at[pl.ds(base, CNT_ROWS_PER_TILE)], dst_v)

    @pl.loop(0, CNT_ROWS_PER_TILE, step=2)
    def _row(r):
      pltpu.async_copy(ones_v, cnt_sh.at[dst_v.at[r]], sem, add=True)
      pltpu.async_copy(ones_v, cnt_sh.at[dst_v.at[r + 1]], sem, add=True)
      pltpu.make_async_copy(ones_v, cnt_sh.at[dst_v.at[r]], sem).wait()
      pltpu.make_async_copy(ones_v, cnt_sh.at[dst_v.at[r + 1]], sem).wait()

    plsc.subcore_barrier()
    pltpu.sync_copy(cnt_sh.at[pl.ds(nb, NODE_ROWS_PER_TILE)],
                    cnt_out.at[c, pl.ds(nb, NODE_ROWS_PER_TILE)])

  return pl.kernel(body, out_type=jax.ShapeDtypeStruct((NC, NP, 16),
                                                       jnp.float32),
                   mesh=mesh, scratch_types=tuple(scratch),
                   compiler_params=pltpu.CompilerParams(
                       use_tc_tiling_on_sc=False))


_sc_cache = {}


def _agg():
  if 'agg' not in _sc_cache:
    _sc_cache['agg'] = _make_agg_kernel()
  return _sc_cache['agg']


def _count():
  if 'cnt' not in _sc_cache:
    _sc_cache['cnt'] = _make_count_kernel()
  return _sc_cache['cnt']


BLK = 2048


def _layer_common(p_ref, cnt_ref, x_ref, aW_ref, ab_ref, lW_ref, lb_ref):
  summed = jnp.concatenate([p_ref[0], p_ref[1]], axis=-1)
  cnt = cnt_ref[0][:, :1] + cnt_ref[1][:, :1]
  agg = summed / jnp.maximum(cnt, 1.0)
  x = jnp.concatenate([x_ref[0], x_ref[1]], axis=-1)
  t = jnp.maximum(
      jnp.dot(agg, aW_ref[...], preferred_element_type=jnp.float32)
      + ab_ref[...], 0.0)
  t = t + jnp.dot(x, lW_ref[...],
                  preferred_element_type=jnp.float32) + lb_ref[...]
  nrm = jnp.sqrt(jnp.sum(t * t, axis=-1, keepdims=True))
  t = t / jnp.maximum(nrm, 1e-12)
  return jnp.maximum(t, 0.0)


def _tc_layer_body(p_ref, cnt_ref, x_ref, aW_ref, ab_ref, lW_ref, lb_ref,
                   o_ref):
  t = _layer_common(p_ref, cnt_ref, x_ref, aW_ref, ab_ref, lW_ref, lb_ref)
  o_ref[0] = t[:, :DH]
  o_ref[1] = t[:, DH:]


def _tc_mlp_body(x_ref, m1W_ref, m1b_ref, m2W_ref, m2b_ref, o_ref):
  h = jnp.concatenate([x_ref[0], x_ref[1]], axis=-1)
  h = jnp.dot(h, m1W_ref[...], preferred_element_type=jnp.float32) \
      + m1b_ref[...]
  h = jnp.dot(h, m2W_ref[...], preferred_element_type=jnp.float32) \
      + m2b_ref[...]
  m = jnp.max(h, axis=-1, keepdims=True)
  e = jnp.exp(h - m)
  o_ref[...] = (h - m) - jnp.log(jnp.sum(e, axis=-1, keepdims=True))


def _full_spec(shape):
  return pl.BlockSpec(shape, lambda i: tuple(0 for _ in shape))


_tc_layer = pl.pallas_call(
    _tc_layer_body,
    grid=(NP // BLK,),
    in_specs=[
        pl.BlockSpec((NC, BLK, DH), lambda i: (0, i, 0)),
        pl.BlockSpec((NC, BLK, 16), lambda i: (0, i, 0)),
        pl.BlockSpec((NC, BLK, DH), lambda i: (0, i, 0)),
        _full_spec((D, D)), _full_spec((1, D)),
        _full_spec((D, D)), _full_spec((1, D)),
    ],
    out_specs=pl.BlockSpec((NC, BLK, DH), lambda i: (0, i, 0)),
    out_shape=jax.ShapeDtypeStruct((NC, NP, DH), jnp.float32),
)

_tc_mlp = pl.pallas_call(
    _tc_mlp_body,
    grid=(NP // BLK,),
    in_specs=[
        pl.BlockSpec((NC, BLK, DH), lambda i: (0, i, 0)),
        _full_spec((D, D)), _full_spec((1, D)),
        _full_spec((D, 64)), _full_spec((1, 64)),
    ],
    out_specs=pl.BlockSpec((BLK, 64), lambda i: (i, 0)),
    out_shape=jax.ShapeDtypeStruct((NP, 64), jnp.float32),
)


def kernel(x, edge_index, batch, lin_W0, lin_b0, agg_W0, agg_b0,
           lin_W1, lin_b1, agg_W1, agg_b1, mp_W1, mp_b1, mp_W2, mp_b2):
  src = edge_index[0]
  dst = edge_index[1]

  xp = jnp.concatenate(
      [x, jnp.zeros((NP - N_NODES, D), jnp.float32)], axis=0)
  xs = jnp.moveaxis(xp.reshape(NP, NC, DH), 1, 0)
  pad = EP - N_EDGES
  srcR = jnp.concatenate([src, jnp.zeros((pad,), jnp.int32)]).reshape(
      IDX_ROWS, 128)
  dstR = jnp.concatenate(
      [dst, jnp.full((pad,), NP - 1, jnp.int32)]).reshape(IDX_ROWS, 128)
  zrow = jnp.zeros((NODE_ROWS_PER_TILE, DH), jnp.float32)
  zcnt = jnp.zeros((NODE_ROWS_PER_TILE, 16), jnp.float32)
  ones = jnp.ones((128, 16), jnp.float32)

  cnt = _count()(dstR, zcnt, ones)

  aggW = jnp.stack([agg_W0, agg_W1])
  aggB = jnp.stack([agg_b0.reshape(1, D), agg_b1.reshape(1, D)])
  linW = jnp.stack([lin_W0, lin_W1])
  linB = jnp.stack([lin_b0.reshape(1, D), lin_b1.reshape(1, D)])

  def layer(i, h):
    sums = _agg()(h, srcR, dstR, zrow)
    return _tc_layer(sums, cnt, h, aggW[i], aggB[i], linW[i], linB[i])

  h = lax.fori_loop(0, 2, layer, xs)

  out = _tc_mlp(h, mp_W1, mp_b1.reshape(1, D), mp_W2, mp_b2.reshape(1, 64))
  return out[:N_NODES]


# speedup vs baseline: 4.2815x; 1.0020x over previous
import jax
import jax.numpy as jnp
from jax import lax
from jax.experimental import pallas as pl
from jax.experimental.pallas import tpu as pltpu
from jax.experimental.pallas import tpu_sc as plsc

N_NODES = 10000
N_EDGES = 320000
D = 128
DH = D // 2

NC = 2
NS = 16

NP = 10240
NODE_ROWS_PER_TILE = NP // NS

EP = 327680
IDX_ROWS = EP // 128
ROWS_PER_TILE = IDX_ROWS // NS
CHUNK_ROWS = 5
N_CHUNKS = ROWS_PER_TILE // CHUNK_ROWS


def _make_agg_kernel():
  mesh = plsc.VectorSubcoreMesh(core_axis_name="c", subcore_axis_name="s",
                                num_cores=NC, num_subcores=NS)
  scratch = [
      pltpu.VMEM((2 * CHUNK_ROWS, 128), jnp.int32),
      pltpu.VMEM((2 * CHUNK_ROWS, 128), jnp.int32),
      pltpu.VMEM((2 * CHUNK_ROWS * 128, DH), jnp.float32),
      pltpu.VMEM_SHARED((NP, DH), jnp.float32),
      pltpu.SemaphoreType.DMA,
      pltpu.SemaphoreType.DMA,
  ]

  def body(x_hbm, src_hbm, dst_hbm, zrow_hbm,
           sum_out, src_v, dst_v, rows_v, acc_sh, sem_g, sem_s):
    c = lax.axis_index("c")
    s = lax.axis_index("s")
    nb = s * NODE_ROWS_PER_TILE
    pltpu.sync_copy(zrow_hbm, acc_sh.at[pl.ds(nb, NODE_ROWS_PER_TILE)])
    plsc.subcore_barrier()

    base = s * ROWS_PER_TILE

    def _row_buf(b, j):
      return rows_v.at[pl.ds((b * CHUNK_ROWS + j) * 128, 128)]

    def _idx_load(k, b):
      r0 = base + k * CHUNK_ROWS
      pltpu.sync_copy(src_hbm.at[pl.ds(r0, CHUNK_ROWS)],
                      src_v.at[pl.ds(b * CHUNK_ROWS, CHUNK_ROWS)])
      pltpu.sync_copy(dst_hbm.at[pl.ds(r0, CHUNK_ROWS)],
                      dst_v.at[pl.ds(b * CHUNK_ROWS, CHUNK_ROWS)])

    def _gather(k, b, issue):
      for j in range(CHUNK_ROWS):
        src = x_hbm.at[c].at[src_v.at[b * CHUNK_ROWS + j]]
        if issue:
          pltpu.async_copy(src, _row_buf(b, j), sem_g)
        else:
          pltpu.make_async_copy(src, _row_buf(b, j), sem_g).wait()

    def _scatter(k, b, issue):
      for j in range(CHUNK_ROWS):
        idx = dst_v.at[b * CHUNK_ROWS + j]
        if issue:
          pltpu.async_copy(_row_buf(b, j), acc_sh.at[idx], sem_s, add=True)
        else:
          pltpu.make_async_copy(_row_buf(b, j), acc_sh.at[idx], sem_s).wait()

    _idx_load(0, 0)
    _gather(0, 0, True)

    @pl.loop(0, N_CHUNKS, step=2)
    def _pipe(g):
      _idx_load(g + 1, 1)
      _gather(g, 0, False)
      _scatter(g, 0, True)

      @pl.when(g > 0)
      def _():
        _scatter(g - 1, 1, False)

      _gather(g + 1, 1, True)
      _gather(g + 1, 1, False)
      _scatter(g + 1, 1, True)
      _scatter(g, 0, False)

      @pl.when(g + 2 < N_CHUNKS)
      def _():
        _idx_load(g + 2, 0)
        _gather(g + 2, 0, True)

    _scatter(N_CHUNKS - 1, 1, False)
    plsc.subcore_barrier()
    pltpu.sync_copy(acc_sh.at[pl.ds(nb, NODE_ROWS_PER_TILE)],
                    sum_out.at[c, pl.ds(nb, NODE_ROWS_PER_TILE)])

  return pl.kernel(body, out_type=jax.ShapeDtypeStruct((NC, NP, DH),
                                                       jnp.float32),
                   mesh=mesh, scratch_types=tuple(scratch),
                   compiler_params=pltpu.CompilerParams(
                       use_tc_tiling_on_sc=False))


CNT_ROWS_PER_TILE = IDX_ROWS // (NC * NS)


def _make_count_kernel():
  mesh = plsc.VectorSubcoreMesh(core_axis_name="c", subcore_axis_name="s",
                                num_cores=NC, num_subcores=NS)
  scratch = [
      pltpu.VMEM((CNT_ROWS_PER_TILE, 128), jnp.int32),
      pltpu.VMEM((128, 16), jnp.float32),
      pltpu.VMEM_SHARED((NP, 16), jnp.float32),
      pltpu.SemaphoreType.DMA,
  ]

  def body(dst_hbm, zcnt_hbm, ones_hbm, cnt_out, dst_v, ones_v, cnt_sh,
           sem):
    c = lax.axis_index("c")
    s = lax.axis_index("s")
    nb = s * NODE_ROWS_PER_TILE
    pltpu.sync_copy(zcnt_hbm, cnt_sh.at[pl.ds(nb, NODE_ROWS_PER_TILE)])
    pltpu.sync_copy(ones_hbm, ones_v)
    plsc.subcore_barrier()

    base = (c * NS + s) * CNT_ROWS_PER_TILE
    pltpu.sync_copy(dst_hbm.at[pl.ds(base, CNT_ROWS_PER_TILE)], dst_v)

    @pl.loop(0, CNT_ROWS_PER_TILE, step=2)
    def _row(r):
      pltpu.async_copy(ones_v, cnt_sh.at[dst_v.at[r]], sem, add=True)
      pltpu.async_copy(ones_v, cnt_sh.at[dst_v.at[r + 1]], sem, add=True)
      pltpu.make_async_copy(ones_v, cnt_sh.at[dst_v.at[r]], sem).wait()
      pltpu.make_async_copy(ones_v, cnt_sh.at[dst_v.at[r + 1]], sem).wait()

    plsc.subcore_barrier()
    pltpu.sync_copy(cnt_sh.at[pl.ds(nb, NODE_ROWS_PER_TILE)],
                    cnt_out.at[c, pl.ds(nb, NODE_ROWS_PER_TILE)])

  return pl.kernel(body, out_type=jax.ShapeDtypeStruct((NC, NP, 16),
                                                       jnp.float32),
                   mesh=mesh, scratch_types=tuple(scratch),
                   compiler_params=pltpu.CompilerParams(
                       use_tc_tiling_on_sc=False))


_sc_cache = {}


def _agg():
  if 'agg' not in _sc_cache:
    _sc_cache['agg'] = _make_agg_kernel()
  return _sc_cache['agg']


def _count():
  if 'cnt' not in _sc_cache:
    _sc_cache['cnt'] = _make_count_kernel()
  return _sc_cache['cnt']


BLK = 2048


def _layer_common(p_ref, cnt_ref, x_ref, aW_ref, ab_ref, lW_ref, lb_ref):
  summed = jnp.concatenate([p_ref[0], p_ref[1]], axis=-1)
  cnt = cnt_ref[0][:, :1] + cnt_ref[1][:, :1]
  agg = summed / jnp.maximum(cnt, 1.0)
  x = jnp.concatenate([x_ref[0], x_ref[1]], axis=-1)
  t = jnp.maximum(
      jnp.dot(agg, aW_ref[...], preferred_element_type=jnp.float32)
      + ab_ref[...], 0.0)
  t = t + jnp.dot(x, lW_ref[...],
                  preferred_element_type=jnp.float32) + lb_ref[...]
  nrm = jnp.sqrt(jnp.sum(t * t, axis=-1, keepdims=True))
  t = t / jnp.maximum(nrm, 1e-12)
  return jnp.maximum(t, 0.0)


def _tc_layer_body(p_ref, cnt_ref, x_ref, aW_ref, ab_ref, lW_ref, lb_ref,
                   o_ref):
  t = _layer_common(p_ref, cnt_ref, x_ref, aW_ref, ab_ref, lW_ref, lb_ref)
  o_ref[0] = t[:, :DH]
  o_ref[1] = t[:, DH:]


def _tc_mlp_body(x_ref, m1W_ref, m1b_ref, m2W_ref, m2b_ref, o_ref):
  h = jnp.concatenate([x_ref[0], x_ref[1]], axis=-1)
  h = jnp.dot(h, m1W_ref[...], preferred_element_type=jnp.float32) \
      + m1b_ref[...]
  h = jnp.dot(h, m2W_ref[...], preferred_element_type=jnp.float32) \
      + m2b_ref[...]
  m = jnp.max(h, axis=-1, keepdims=True)
  e = jnp.exp(h - m)
  o_ref[...] = (h - m) - jnp.log(jnp.sum(e, axis=-1, keepdims=True))


def _full_spec(shape):
  return pl.BlockSpec(shape, lambda i: tuple(0 for _ in shape))


_tc_layer = pl.pallas_call(
    _tc_layer_body,
    grid=(NP // BLK,),
    in_specs=[
        pl.BlockSpec((NC, BLK, DH), lambda i: (0, i, 0)),
        pl.BlockSpec((NC, BLK, 16), lambda i: (0, i, 0)),
        pl.BlockSpec((NC, BLK, DH), lambda i: (0, i, 0)),
        _full_spec((D, D)), _full_spec((1, D)),
        _full_spec((D, D)), _full_spec((1, D)),
    ],
    out_specs=pl.BlockSpec((NC, BLK, DH), lambda i: (0, i, 0)),
    out_shape=jax.ShapeDtypeStruct((NC, NP, DH), jnp.float32),
)

_tc_mlp = pl.pallas_call(
    _tc_mlp_body,
    grid=(NP // BLK,),
    in_specs=[
        pl.BlockSpec((NC, BLK, DH), lambda i: (0, i, 0)),
        _full_spec((D, D)), _full_spec((1, D)),
        _full_spec((D, 64)), _full_spec((1, 64)),
    ],
    out_specs=pl.BlockSpec((BLK, 64), lambda i: (i, 0)),
    out_shape=jax.ShapeDtypeStruct((NP, 64), jnp.float32),
)


def kernel(x, edge_index, batch, lin_W0, lin_b0, agg_W0, agg_b0,
           lin_W1, lin_b1, agg_W1, agg_b1, mp_W1, mp_b1, mp_W2, mp_b2):
  src = edge_index[0]
  dst = edge_index[1]

  xp = jnp.concatenate(
      [x, jnp.zeros((NP - N_NODES, D), jnp.float32)], axis=0)
  xs = jnp.moveaxis(xp.reshape(NP, NC, DH), 1, 0)
  pad = EP - N_EDGES
  srcR = jnp.concatenate([src, jnp.zeros((pad,), jnp.int32)]).reshape(
      IDX_ROWS, 128)
  dstR = jnp.concatenate(
      [dst, jnp.full((pad,), NP - 1, jnp.int32)]).reshape(IDX_ROWS, 128)
  zrow = jnp.zeros((NODE_ROWS_PER_TILE, DH), jnp.float32)
  zcnt = jnp.zeros((NODE_ROWS_PER_TILE, 16), jnp.float32)
  ones = jnp.ones((128, 16), jnp.float32)

  cnt = _count()(dstR, zcnt, ones)

  aggW = jnp.stack([agg_W0, agg_W1])
  aggB = jnp.stack([agg_b0.reshape(1, D), agg_b1.reshape(1, D)])
  linW = jnp.stack([lin_W0, lin_W1])
  linB = jnp.stack([lin_b0.reshape(1, D), lin_b1.reshape(1, D)])

  def layer(i, h):
    sums = _agg()(h, srcR, dstR, zrow)
    return _tc_layer(sums, cnt, h, aggW[i], aggB[i], linW[i], linB[i])

  h = layer(1, layer(0, xs))

  out = _tc_mlp(h, mp_W1, mp_b1.reshape(1, D), mp_W2, mp_b2.reshape(1, 64))
  return out[:N_NODES]
